# baseline (speedup 1.0000x reference)
import functools

import jax
import jax.numpy as jnp
from jax import lax
from jax.experimental import pallas as pl
from jax.experimental.pallas import tpu as pltpu
from jax.experimental.pallas import tpu_sc as plsc

N, E, D, H, L, OUT = 10000, 320000, 128, 256, 64, 128
EPS = 1e-5

NC, NS = 2, 16
NW = NC * NS
LANES = 16

_MESH = plsc.VectorSubcoreMesh(core_axis_name="c", subcore_axis_name="s")


def _fill_1d(ref, length, value, dtype):
    vec = jnp.full((LANES,), value, dtype)

    @pl.loop(0, length, step=LANES)
    def _(i):
        ref[pl.ds(i, LANES)] = vec


def _sc_degrees(src, dst):
    epw = E // NS
    ept = E // NW
    ch = 1000
    zslice = 2000

    @functools.partial(
        pl.kernel,
        out_type=jax.ShapeDtypeStruct((NC * 2 * N,), jnp.float32),
        mesh=_MESH,
        scratch_types=[
            pltpu.VMEM((ch,), jnp.int32),
            pltpu.VMEM((ch,), jnp.float32),
            pltpu.VMEM((zslice,), jnp.float32),
            pltpu.VMEM_SHARED((2 * N,), jnp.float32),
        ],
    )
    def k(src_hbm, dst_hbm, out_hbm, idx_v, ones_v, zero_v, acc_sh):
        c = lax.axis_index("c")
        s = lax.axis_index("s")
        wid = c * NS + s
        _fill_1d(ones_v, ch, 1.0, jnp.float32)

        @pl.when(s < 10)
        def _():
            _fill_1d(zero_v, zslice, 0.0, jnp.float32)
            pltpu.sync_copy(zero_v, acc_sh.at[pl.ds(s * zslice, zslice)])

        plsc.subcore_barrier()

        base = wid * ept

        @pl.loop(0, ept, step=ch)
        def _(e0):
            pltpu.sync_copy(src_hbm.at[pl.ds(base + e0, ch)], idx_v)
            pltpu.sync_copy(ones_v, acc_sh.at[idx_v], add=True)
            pltpu.sync_copy(dst_hbm.at[pl.ds(base + e0, ch)], idx_v)

            @pl.loop(0, ch, step=LANES)
            def _(i):
                idx_v[pl.ds(i, LANES)] = idx_v[pl.ds(i, LANES)] + N

            pltpu.sync_copy(ones_v, acc_sh.at[idx_v], add=True)

        plsc.subcore_barrier()

        @pl.when(s < 10)
        def _():
            pltpu.sync_copy(acc_sh.at[pl.ds(s * zslice, zslice)], zero_v)
            pltpu.sync_copy(zero_v, out_hbm.at[pl.ds(c * 2 * N + s * zslice, zslice)])

    return k(src, dst)


NPAD = 10048
CH = 184


def _sc_message(tab, srcp, dstp, zeros_nw, feature_split, cpt):
    w = 128
    ept = cpt * CH

    @functools.partial(
        pl.kernel,
        out_type=jax.ShapeDtypeStruct((NC, N, w), jnp.float32),
        mesh=_MESH,
        scratch_types=[
            pltpu.VMEM((CH,), jnp.int32),
            pltpu.VMEM((CH,), jnp.int32),
            pltpu.VMEM((CH,), jnp.int32),
            pltpu.VMEM((CH,), jnp.int32),
            pltpu.VMEM((CH, w), jnp.float32),
            pltpu.VMEM((CH, w), jnp.float32),
            pltpu.VMEM_SHARED((NPAD, w), jnp.float32),
            pltpu.SemaphoreType.DMA,
            pltpu.SemaphoreType.DMA,
        ],
    )
    def k(tab_hbm, src_hbm, dst_hbm, zeros_hbm, out_hbm,
          sidx, didx, sidx2, didx2, m0, m1, acc_sh, gsem, ssem):
        c = lax.axis_index("c")
        s = lax.axis_index("s")
        if feature_split:
            tabref = tab_hbm.at[c]
        else:
            tabref = tab_hbm

        fs = 80

        def zchunk(row0, sz):
            pltpu.sync_copy(zeros_hbm.at[pl.ds(row0, sz)], m0.at[pl.ds(0, sz)])
            pltpu.sync_copy(m0.at[pl.ds(0, sz)], acc_sh.at[pl.ds(row0, sz)])

        @pl.when(s < 15)
        def _():
            @pl.loop(0, 640, step=fs)
            def _(r):
                zchunk(s * 640 + r, fs)

        @pl.when(s == 15)
        def _():
            @pl.loop(0, 400, step=fs)
            def _(r):
                zchunk(9600 + r, fs)

        plsc.subcore_barrier()

        base = (s if feature_split else c * NS + s) * ept

        def load_idx(e0, sref, dref):
            pltpu.sync_copy(src_hbm.at[pl.ds(e0, CH)], sref)
            pltpu.sync_copy(dst_hbm.at[pl.ds(e0, CH)], dref)

        def gstart(sref, mref):
            pltpu.async_copy(tabref.at[sref], mref, gsem)

        def gwait(sref, mref):
            pltpu.make_async_copy(tabref.at[sref], mref, gsem).wait()

        def sstart(mref, dref):
            pltpu.async_copy(mref, acc_sh.at[dref], ssem, add=True)

        def swait(mref, dref):
            pltpu.make_async_copy(mref, acc_sh.at[dref], ssem).wait()

        load_idx(base, sidx, didx)
        gstart(sidx, m0)
        load_idx(base + CH, sidx2, didx2)
        gwait(sidx, m0)
        gstart(sidx2, m1)
        sstart(m0, didx)

        @pl.loop(2, cpt, step=2)
        def _(k_):
            e0 = base + k_ * CH
            swait(m0, didx)
            load_idx(e0, sidx, didx)
            gwait(sidx2, m1)
            gstart(sidx, m0)
            sstart(m1, didx2)
            swait(m1, didx2)
            load_idx(e0 + CH, sidx2, didx2)
            gwait(sidx, m0)
            gstart(sidx2, m1)
            sstart(m0, didx)

        swait(m0, didx)
        gwait(sidx2, m1)
        sstart(m1, didx2)
        swait(m1, didx2)

        plsc.subcore_barrier()

        def fchunk(row0, sz):
            pltpu.sync_copy(acc_sh.at[pl.ds(row0, sz)], m0.at[pl.ds(0, sz)])
            pltpu.sync_copy(m0.at[pl.ds(0, sz)], out_hbm.at[c, pl.ds(row0, sz)])

        @pl.when(s < 15)
        def _():
            @pl.loop(0, 640, step=fs)
            def _(r):
                fchunk(s * 640 + r, fs)

        @pl.when(s == 15)
        def _():
            @pl.loop(0, 400, step=fs)
            def _(r):
                fchunk(9600 + r, fs)

    return k(tab, srcp, dstp, zeros_nw)


def _pad_edges(src, dst, nranges, cpt):
    per = E // nranges
    pad = cpt * CH - per
    s2 = jnp.pad(src.reshape(nranges, per), ((0, 0), (0, pad))).reshape(-1)
    dump = N + 16 + jnp.arange(nranges, dtype=jnp.int32)[:, None]
    d2 = jnp.concatenate(
        [dst.reshape(nranges, per),
         jnp.broadcast_to(dump, (nranges, pad))], axis=1).reshape(-1)
    return s2, d2


BN = 2000


def _tc_norm_scale(degp, x):
    def body(degp_ref, x_ref, ns_ref, nd_ref, xs_ref):
        deg_s = degp_ref[0] + degp_ref[2]
        deg_d = degp_ref[1] + degp_ref[3]
        ns_full = lax.rsqrt(jnp.maximum(deg_s, 1.0))
        nd_full = lax.rsqrt(jnp.maximum(deg_d, 1.0))
        ns_ref[...] = ns_full
        nd_ref[...] = nd_full
        xs_ref[...] = x_ref[...] * ns_full

    grid = N // BN
    return pl.pallas_call(
        body,
        grid=(grid,),
        in_specs=[
            pl.BlockSpec((4, BN, 1), lambda i: (0, i, 0)),
            pl.BlockSpec((BN, D), lambda i: (i, 0)),
        ],
        out_specs=[
            pl.BlockSpec((BN, 1), lambda i: (i, 0)),
            pl.BlockSpec((BN, 1), lambda i: (i, 0)),
            pl.BlockSpec((BN, D), lambda i: (i, 0)),
        ],
        out_shape=[
            jax.ShapeDtypeStruct((N, 1), jnp.float32),
            jax.ShapeDtypeStruct((N, 1), jnp.float32),
            jax.ShapeDtypeStruct((N, D), jnp.float32),
        ],
    )(degp, x)


def _tc_gcn1(aggp, nd, ns, g1W, g1b):
    def body(aggp_ref, nd_ref, ns_ref, w_ref, b_ref, hs_ref):
        agg = (aggp_ref[0] + aggp_ref[1]) * nd_ref[...]
        h = jnp.maximum(
            jnp.dot(agg, w_ref[...], preferred_element_type=jnp.float32)
            + b_ref[...],
            0.0,
        )
        hs = h * ns_ref[...]
        hs_ref[0] = hs[:, : H // 2]
        hs_ref[1] = hs[:, H // 2 :]

    grid = N // BN
    return pl.pallas_call(
        body,
        grid=(grid,),
        in_specs=[
            pl.BlockSpec((NC, BN, D), lambda i: (0, i, 0)),
            pl.BlockSpec((BN, 1), lambda i: (i, 0)),
            pl.BlockSpec((BN, 1), lambda i: (i, 0)),
            pl.BlockSpec((D, H), lambda i: (0, 0)),
            pl.BlockSpec((H,), lambda i: (0,)),
        ],
        out_specs=pl.BlockSpec((NC, BN, H // 2), lambda i: (0, i, 0)),
        out_shape=jax.ShapeDtypeStruct((NC, N, H // 2), jnp.float32),
    )(aggp, nd, ns, g1W, g1b)


def _ln(x, g, b):
    m = jnp.mean(x, axis=-1, keepdims=True)
    v = jnp.mean(jnp.square(x - m), axis=-1, keepdims=True)
    return (x - m) * lax.rsqrt(v + EPS) * g + b


def _tc_mlp(agg2p, nd, ws):
    (g2W, g2b, e1W, e1b, e1g, e1be, e2W, e2b, e2g, e2be,
     d1W, d1b, d1g, d1be, d2W, d2b, d2g, d2be,
     m1W, m1b, m1g, m1be, m2W, m2b) = ws

    def body(agg_ref, nd_ref, g2W_, g2b_, e1W_, e1b_, e1g_, e1be_, e2W_,
             e2b_, e2g_, e2be_, d1W_, d1b_, d1g_, d1be_, d2W_, d2b_, d2g_,
             d2be_, m1W_, m1b_, m1g_, m1be_, m2W_, m2b_, out_ref):
        agg = jnp.concatenate([agg_ref[0], agg_ref[1]], axis=1)
        agg = agg * nd_ref[...]

        def mm(a, w, b):
            return jnp.dot(a, w[...], preferred_element_type=jnp.float32) + b[...]

        h = jnp.maximum(mm(agg, g2W_, g2b_), 0.0)
        h = _ln(jnp.maximum(mm(h, e1W_, e1b_), 0.0), e1g_[...], e1be_[...])
        z = _ln(jnp.maximum(mm(h, e2W_, e2b_), 0.0), e2g_[...], e2be_[...])
        h = _ln(jnp.maximum(mm(z, d1W_, d1b_), 0.0), d1g_[...], d1be_[...])
        h = _ln(jnp.maximum(mm(h, d2W_, d2b_), 0.0), d2g_[...], d2be_[...])
        h = _ln(jnp.maximum(mm(h, m1W_, m1b_), 0.0), m1g_[...], m1be_[...])
        out_ref[...] = jnp.maximum(mm(h, m2W_, m2b_), 0.0)

    grid = N // BN
    w_specs = []
    for warr in ws:
        if warr.ndim == 2:
            w_specs.append(pl.BlockSpec(warr.shape, lambda i: (0, 0)))
        else:
            w_specs.append(pl.BlockSpec(warr.shape, lambda i: (0,)))
    return pl.pallas_call(
        body,
        grid=(grid,),
        in_specs=[
            pl.BlockSpec((NC, BN, H // 2), lambda i: (0, i, 0)),
            pl.BlockSpec((BN, 1), lambda i: (i, 0)),
        ] + w_specs,
        out_specs=pl.BlockSpec((BN, OUT), lambda i: (i, 0)),
        out_shape=jax.ShapeDtypeStruct((N, OUT), jnp.float32),
    )(agg2p, nd, *ws)


def kernel(x, edge_index, g1W, g1b, g2W, g2b, e1W, e1b, e1g, e1be, e2W, e2b,
           e2g, e2be, d1W, d1b, d1g, d1be, d2W, d2b, d2g, d2be, m1W, m1b,
           m1g, m1be, m2W, m2b):
    src = edge_index[0]
    dst = edge_index[1]
    degp = _sc_degrees(src, dst).reshape(4, N, 1)
    ns, nd, xs = _tc_norm_scale(degp, x)
    zeros128 = jnp.zeros((N, 128), jnp.float32)
    src1, dst1 = _pad_edges(src, dst, NW, 56)
    src2, dst2 = _pad_edges(src, dst, NS, 110)
    aggp = _sc_message(xs, src1, dst1, zeros128, False, 56)
    hs = _tc_gcn1(aggp, nd, ns, g1W, g1b)
    agg2p = _sc_message(hs, src2, dst2, zeros128, True, 110)
    ws = (g2W, g2b, e1W, e1b, e1g, e1be, e2W, e2b, e2g, e2be,
          d1W, d1b, d1g, d1be, d2W, d2b, d2g, d2be,
          m1W, m1b, m1g, m1be, m2W, m2b)
    return _tc_mlp(agg2p, nd, ws)


# speedup vs baseline: 1.4218x; 1.4218x over previous
import functools

import jax
import jax.numpy as jnp
from jax import lax
from jax.experimental import pallas as pl
from jax.experimental.pallas import tpu as pltpu
from jax.experimental.pallas import tpu_sc as plsc

N, E, D, H, L, OUT = 10000, 320000, 128, 256, 64, 128
EPS = 1e-5

NC, NS = 2, 16
NW = NC * NS
LANES = 16

_MESH = plsc.VectorSubcoreMesh(core_axis_name="c", subcore_axis_name="s")


def _fill_1d(ref, length, value, dtype):
    vec = jnp.full((LANES,), value, dtype)

    @pl.loop(0, length, step=LANES)
    def _(i):
        ref[pl.ds(i, LANES)] = vec


def _sc_degrees(src, dst):
    epw = E // NS
    ept = E // NW
    ch = 1000
    zslice = 2000

    @functools.partial(
        pl.kernel,
        out_type=jax.ShapeDtypeStruct((NC * 2 * N,), jnp.float32),
        mesh=_MESH,
        scratch_types=[
            pltpu.VMEM((ch,), jnp.int32),
            pltpu.VMEM((ch,), jnp.float32),
            pltpu.VMEM((zslice,), jnp.float32),
            pltpu.VMEM_SHARED((2 * N,), jnp.float32),
        ],
    )
    def k(src_hbm, dst_hbm, out_hbm, idx_v, ones_v, zero_v, acc_sh):
        c = lax.axis_index("c")
        s = lax.axis_index("s")
        wid = c * NS + s
        _fill_1d(ones_v, ch, 1.0, jnp.float32)

        @pl.when(s < 10)
        def _():
            _fill_1d(zero_v, zslice, 0.0, jnp.float32)
            pltpu.sync_copy(zero_v, acc_sh.at[pl.ds(s * zslice, zslice)])

        plsc.subcore_barrier()

        base = wid * ept

        @pl.loop(0, ept, step=ch)
        def _(e0):
            pltpu.sync_copy(src_hbm.at[pl.ds(base + e0, ch)], idx_v)
            pltpu.sync_copy(ones_v, acc_sh.at[idx_v], add=True)
            pltpu.sync_copy(dst_hbm.at[pl.ds(base + e0, ch)], idx_v)

            @pl.loop(0, ch, step=LANES)
            def _(i):
                idx_v[pl.ds(i, LANES)] = idx_v[pl.ds(i, LANES)] + N

            pltpu.sync_copy(ones_v, acc_sh.at[idx_v], add=True)

        plsc.subcore_barrier()

        @pl.when(s < 10)
        def _():
            pltpu.sync_copy(acc_sh.at[pl.ds(s * zslice, zslice)], zero_v)
            pltpu.sync_copy(zero_v, out_hbm.at[pl.ds(c * 2 * N + s * zslice, zslice)])

    return k(src, dst)


NPAD = 10048
CH = 200


def _sc_message(tab, srcp, dstp, zeros_nw, feature_split, cpt):
    w = 128
    ept = cpt * CH

    @functools.partial(
        pl.kernel,
        out_type=jax.ShapeDtypeStruct((NC, N, w), jnp.float32),
        mesh=_MESH,
        scratch_types=[
            pltpu.VMEM((CH,), jnp.int32),
            pltpu.VMEM((CH,), jnp.int32),
            pltpu.VMEM((CH, w), jnp.float32),
            pltpu.VMEM_SHARED((NPAD, w), jnp.float32),
        ],
    )
    def k(tab_hbm, src_hbm, dst_hbm, zeros_hbm, out_hbm,
          sidx, didx, m0, acc_sh):
        c = lax.axis_index("c")
        s = lax.axis_index("s")
        if feature_split:
            tabref = tab_hbm.at[c]
        else:
            tabref = tab_hbm

        fs = 80

        def zchunk(row0, sz):
            pltpu.sync_copy(zeros_hbm.at[pl.ds(row0, sz)], m0.at[pl.ds(0, sz)])
            pltpu.sync_copy(m0.at[pl.ds(0, sz)], acc_sh.at[pl.ds(row0, sz)])

        @pl.when(s < 15)
        def _():
            @pl.loop(0, 640, step=fs)
            def _(r):
                zchunk(s * 640 + r, fs)

        @pl.when(s == 15)
        def _():
            @pl.loop(0, 400, step=fs)
            def _(r):
                zchunk(9600 + r, fs)

        plsc.subcore_barrier()

        base = (s if feature_split else c * NS + s) * ept

        @pl.loop(0, ept, step=CH)
        def _(e0):
            pltpu.sync_copy(src_hbm.at[pl.ds(base + e0, CH)], sidx)
            pltpu.sync_copy(dst_hbm.at[pl.ds(base + e0, CH)], didx)
            pltpu.sync_copy(tabref.at[sidx], m0)
            pltpu.sync_copy(m0, acc_sh.at[didx], add=True)

        plsc.subcore_barrier()

        def fchunk(row0, sz):
            pltpu.sync_copy(acc_sh.at[pl.ds(row0, sz)], m0.at[pl.ds(0, sz)])
            pltpu.sync_copy(m0.at[pl.ds(0, sz)], out_hbm.at[c, pl.ds(row0, sz)])

        @pl.when(s < 15)
        def _():
            @pl.loop(0, 640, step=fs)
            def _(r):
                fchunk(s * 640 + r, fs)

        @pl.when(s == 15)
        def _():
            @pl.loop(0, 400, step=fs)
            def _(r):
                fchunk(9600 + r, fs)

    return k(tab, srcp, dstp, zeros_nw)


def _pad_edges(src, dst, nranges, cpt):
    per = E // nranges
    pad = cpt * CH - per
    s2 = jnp.pad(src.reshape(nranges, per), ((0, 0), (0, pad))).reshape(-1)
    dump = N + 16 + jnp.arange(nranges, dtype=jnp.int32)[:, None]
    d2 = jnp.concatenate(
        [dst.reshape(nranges, per),
         jnp.broadcast_to(dump, (nranges, pad))], axis=1).reshape(-1)
    return s2, d2


BN = 2000


def _tc_norm_scale(degp, x):
    def body(degp_ref, x_ref, ns_ref, nd_ref, xs_ref):
        deg_s = degp_ref[0] + degp_ref[2]
        deg_d = degp_ref[1] + degp_ref[3]
        ns_full = lax.rsqrt(jnp.maximum(deg_s, 1.0))
        nd_full = lax.rsqrt(jnp.maximum(deg_d, 1.0))
        ns_ref[...] = ns_full
        nd_ref[...] = nd_full
        xs_ref[...] = x_ref[...] * ns_full

    grid = N // BN
    return pl.pallas_call(
        body,
        grid=(grid,),
        in_specs=[
            pl.BlockSpec((4, BN, 1), lambda i: (0, i, 0)),
            pl.BlockSpec((BN, D), lambda i: (i, 0)),
        ],
        out_specs=[
            pl.BlockSpec((BN, 1), lambda i: (i, 0)),
            pl.BlockSpec((BN, 1), lambda i: (i, 0)),
            pl.BlockSpec((BN, D), lambda i: (i, 0)),
        ],
        out_shape=[
            jax.ShapeDtypeStruct((N, 1), jnp.float32),
            jax.ShapeDtypeStruct((N, 1), jnp.float32),
            jax.ShapeDtypeStruct((N, D), jnp.float32),
        ],
    )(degp, x)


def _tc_gcn1(aggp, nd, ns, g1W, g1b):
    def body(aggp_ref, nd_ref, ns_ref, w_ref, b_ref, hs_ref):
        agg = (aggp_ref[0] + aggp_ref[1]) * nd_ref[...]
        h = jnp.maximum(
            jnp.dot(agg, w_ref[...], preferred_element_type=jnp.float32)
            + b_ref[...],
            0.0,
        )
        hs = h * ns_ref[...]
        hs_ref[0] = hs[:, : H // 2]
        hs_ref[1] = hs[:, H // 2 :]

    grid = N // BN
    return pl.pallas_call(
        body,
        grid=(grid,),
        in_specs=[
            pl.BlockSpec((NC, BN, D), lambda i: (0, i, 0)),
            pl.BlockSpec((BN, 1), lambda i: (i, 0)),
            pl.BlockSpec((BN, 1), lambda i: (i, 0)),
            pl.BlockSpec((D, H), lambda i: (0, 0)),
            pl.BlockSpec((H,), lambda i: (0,)),
        ],
        out_specs=pl.BlockSpec((NC, BN, H // 2), lambda i: (0, i, 0)),
        out_shape=jax.ShapeDtypeStruct((NC, N, H // 2), jnp.float32),
    )(aggp, nd, ns, g1W, g1b)


def _ln(x, g, b):
    m = jnp.mean(x, axis=-1, keepdims=True)
    v = jnp.mean(jnp.square(x - m), axis=-1, keepdims=True)
    return (x - m) * lax.rsqrt(v + EPS) * g + b


def _tc_mlp(agg2p, nd, ws):
    (g2W, g2b, e1W, e1b, e1g, e1be, e2W, e2b, e2g, e2be,
     d1W, d1b, d1g, d1be, d2W, d2b, d2g, d2be,
     m1W, m1b, m1g, m1be, m2W, m2b) = ws

    def body(agg_ref, nd_ref, g2W_, g2b_, e1W_, e1b_, e1g_, e1be_, e2W_,
             e2b_, e2g_, e2be_, d1W_, d1b_, d1g_, d1be_, d2W_, d2b_, d2g_,
             d2be_, m1W_, m1b_, m1g_, m1be_, m2W_, m2b_, out_ref):
        agg = jnp.concatenate([agg_ref[0], agg_ref[1]], axis=1)
        agg = agg * nd_ref[...]

        def mm(a, w, b):
            return jnp.dot(a, w[...], preferred_element_type=jnp.float32) + b[...]

        h = jnp.maximum(mm(agg, g2W_, g2b_), 0.0)
        h = _ln(jnp.maximum(mm(h, e1W_, e1b_), 0.0), e1g_[...], e1be_[...])
        z = _ln(jnp.maximum(mm(h, e2W_, e2b_), 0.0), e2g_[...], e2be_[...])
        h = _ln(jnp.maximum(mm(z, d1W_, d1b_), 0.0), d1g_[...], d1be_[...])
        h = _ln(jnp.maximum(mm(h, d2W_, d2b_), 0.0), d2g_[...], d2be_[...])
        h = _ln(jnp.maximum(mm(h, m1W_, m1b_), 0.0), m1g_[...], m1be_[...])
        out_ref[...] = jnp.maximum(mm(h, m2W_, m2b_), 0.0)

    grid = N // BN
    w_specs = []
    for warr in ws:
        if warr.ndim == 2:
            w_specs.append(pl.BlockSpec(warr.shape, lambda i: (0, 0)))
        else:
            w_specs.append(pl.BlockSpec(warr.shape, lambda i: (0,)))
    return pl.pallas_call(
        body,
        grid=(grid,),
        in_specs=[
            pl.BlockSpec((NC, BN, H // 2), lambda i: (0, i, 0)),
            pl.BlockSpec((BN, 1), lambda i: (i, 0)),
        ] + w_specs,
        out_specs=pl.BlockSpec((BN, OUT), lambda i: (i, 0)),
        out_shape=jax.ShapeDtypeStruct((N, OUT), jnp.float32),
    )(agg2p, nd, *ws)


def kernel(x, edge_index, g1W, g1b, g2W, g2b, e1W, e1b, e1g, e1be, e2W, e2b,
           e2g, e2be, d1W, d1b, d1g, d1be, d2W, d2b, d2g, d2be, m1W, m1b,
           m1g, m1be, m2W, m2b):
    src = edge_index[0]
    dst = edge_index[1]
    degp = _sc_degrees(src, dst).reshape(4, N, 1)
    ns, nd, xs = _tc_norm_scale(degp, x)
    zeros128 = jnp.zeros((N, 128), jnp.float32)
    src1, dst1 = _pad_edges(src, dst, NW, 50)
    src2, dst2 = _pad_edges(src, dst, NS, 100)
    aggp = _sc_message(xs, src1, dst1, zeros128, False, 50)
    hs = _tc_gcn1(aggp, nd, ns, g1W, g1b)
    agg2p = _sc_message(hs, src2, dst2, zeros128, True, 100)
    ws = (g2W, g2b, e1W, e1b, e1g, e1be, e2W, e2b, e2g, e2be,
          d1W, d1b, d1g, d1be, d2W, d2b, d2g, d2be,
          m1W, m1b, m1g, m1be, m2W, m2b)
    return _tc_mlp(agg2p, nd, ws)


# speedup vs baseline: 1.5561x; 1.0945x over previous
import functools

import jax
import jax.numpy as jnp
from jax import lax
from jax.experimental import pallas as pl
from jax.experimental.pallas import tpu as pltpu
from jax.experimental.pallas import tpu_sc as plsc

N, E, D, H, L, OUT = 10000, 320000, 128, 256, 64, 128
EPS = 1e-5

NC, NS = 2, 16
NW = NC * NS
LANES = 16

_MESH = plsc.VectorSubcoreMesh(core_axis_name="c", subcore_axis_name="s")


def _fill_1d(ref, length, value, dtype):
    vec = jnp.full((LANES,), value, dtype)

    @pl.loop(0, length, step=LANES)
    def _(i):
        ref[pl.ds(i, LANES)] = vec


def _sc_degrees(src, dst):
    epw = E // NS
    ept = E // NW
    ch = 1000
    zslice = 2000

    @functools.partial(
        pl.kernel,
        out_type=jax.ShapeDtypeStruct((NC * 2 * N,), jnp.float32),
        mesh=_MESH,
        scratch_types=[
            pltpu.VMEM((ch,), jnp.int32),
            pltpu.VMEM((ch,), jnp.float32),
            pltpu.VMEM((zslice,), jnp.float32),
            pltpu.VMEM_SHARED((2 * N,), jnp.float32),
        ],
    )
    def k(src_hbm, dst_hbm, out_hbm, idx_v, ones_v, zero_v, acc_sh):
        c = lax.axis_index("c")
        s = lax.axis_index("s")
        wid = c * NS + s
        _fill_1d(ones_v, ch, 1.0, jnp.float32)

        @pl.when(s < 10)
        def _():
            _fill_1d(zero_v, zslice, 0.0, jnp.float32)
            pltpu.sync_copy(zero_v, acc_sh.at[pl.ds(s * zslice, zslice)])

        plsc.subcore_barrier()

        base = wid * ept

        @pl.loop(0, ept, step=ch)
        def _(e0):
            pltpu.sync_copy(src_hbm.at[pl.ds(base + e0, ch)], idx_v)
            pltpu.sync_copy(ones_v, acc_sh.at[idx_v], add=True)
            pltpu.sync_copy(dst_hbm.at[pl.ds(base + e0, ch)], idx_v)

            @pl.loop(0, ch, step=LANES)
            def _(i):
                idx_v[pl.ds(i, LANES)] = idx_v[pl.ds(i, LANES)] + N

            pltpu.sync_copy(ones_v, acc_sh.at[idx_v], add=True)

        plsc.subcore_barrier()

        @pl.when(s < 10)
        def _():
            pltpu.sync_copy(acc_sh.at[pl.ds(s * zslice, zslice)], zero_v)
            pltpu.sync_copy(zero_v, out_hbm.at[pl.ds(c * 2 * N + s * zslice, zslice)])

    return k(src, dst)


NPAD = 10048
CH = 200


def _sc_message(tab, srcp, zeros_nw, feature_split, cpt):
    w = 128
    ept = cpt * CH

    @functools.partial(
        pl.kernel,
        out_type=jax.ShapeDtypeStruct((NC, N, w), jnp.float32),
        mesh=_MESH,
        scratch_types=[
            pltpu.VMEM((2 * CH,), jnp.int32),
            pltpu.VMEM((CH, w), jnp.float32),
            pltpu.VMEM_SHARED((NPAD, w), jnp.float32),
        ],
    )
    def k(tab_hbm, edge_hbm, zeros_hbm, out_hbm, eidx, m0, acc_sh):
        c = lax.axis_index("c")
        s = lax.axis_index("s")
        if feature_split:
            tabref = tab_hbm.at[c]
        else:
            tabref = tab_hbm

        fs = 80

        def zchunk(row0, sz):
            pltpu.sync_copy(zeros_hbm.at[pl.ds(row0, sz)], m0.at[pl.ds(0, sz)])
            pltpu.sync_copy(m0.at[pl.ds(0, sz)], acc_sh.at[pl.ds(row0, sz)])

        @pl.when(s < 15)
        def _():
            @pl.loop(0, 640, step=fs)
            def _(r):
                zchunk(s * 640 + r, fs)

        @pl.when(s == 15)
        def _():
            @pl.loop(0, 400, step=fs)
            def _(r):
                zchunk(9600 + r, fs)

        plsc.subcore_barrier()

        base = (s if feature_split else c * NS + s) * (2 * ept)

        @pl.loop(0, 2 * ept, step=2 * CH)
        def _(e0):
            pltpu.sync_copy(edge_hbm.at[pl.ds(base + e0, 2 * CH)], eidx)
            pltpu.sync_copy(tabref.at[eidx.at[pl.ds(0, CH)]], m0)
            pltpu.sync_copy(m0, acc_sh.at[eidx.at[pl.ds(CH, CH)]], add=True)

        plsc.subcore_barrier()

        def fchunk(row0, sz):
            pltpu.sync_copy(acc_sh.at[pl.ds(row0, sz)], m0.at[pl.ds(0, sz)])
            pltpu.sync_copy(m0.at[pl.ds(0, sz)], out_hbm.at[c, pl.ds(row0, sz)])

        @pl.when(s < 15)
        def _():
            @pl.loop(0, 640, step=fs)
            def _(r):
                fchunk(s * 640 + r, fs)

        @pl.when(s == 15)
        def _():
            @pl.loop(0, 400, step=fs)
            def _(r):
                fchunk(9600 + r, fs)

    return k(tab, srcp, zeros_nw)


def _pad_edges(src, dst, nranges, cpt):
    per = E // nranges
    pad = cpt * CH - per
    s2 = jnp.pad(src.reshape(nranges, per), ((0, 0), (0, pad)))
    dump = N + 16 + jnp.arange(nranges, dtype=jnp.int32)[:, None]
    d2 = jnp.concatenate(
        [dst.reshape(nranges, per),
         jnp.broadcast_to(dump, (nranges, pad))], axis=1)
    inter = jnp.stack([s2.reshape(nranges, cpt, CH),
                       d2.reshape(nranges, cpt, CH)], axis=2)
    return inter.reshape(-1)


BN = 2000


def _tc_norm_scale(degp, x):
    def body(degp_ref, x_ref, ns_ref, nd_ref, xs_ref):
        deg_s = degp_ref[0] + degp_ref[2]
        deg_d = degp_ref[1] + degp_ref[3]
        ns_full = lax.rsqrt(jnp.maximum(deg_s, 1.0))
        nd_full = lax.rsqrt(jnp.maximum(deg_d, 1.0))
        ns_ref[...] = ns_full
        nd_ref[...] = nd_full
        xs_ref[...] = x_ref[...] * ns_full

    grid = N // BN
    return pl.pallas_call(
        body,
        grid=(grid,),
        in_specs=[
            pl.BlockSpec((4, BN, 1), lambda i: (0, i, 0)),
            pl.BlockSpec((BN, D), lambda i: (i, 0)),
        ],
        out_specs=[
            pl.BlockSpec((BN, 1), lambda i: (i, 0)),
            pl.BlockSpec((BN, 1), lambda i: (i, 0)),
            pl.BlockSpec((BN, D), lambda i: (i, 0)),
        ],
        out_shape=[
            jax.ShapeDtypeStruct((N, 1), jnp.float32),
            jax.ShapeDtypeStruct((N, 1), jnp.float32),
            jax.ShapeDtypeStruct((N, D), jnp.float32),
        ],
    )(degp, x)


def _tc_gcn1(aggp, nd, ns, g1W, g1b):
    def body(aggp_ref, nd_ref, ns_ref, w_ref, b_ref, hs_ref):
        agg = (aggp_ref[0] + aggp_ref[1]) * nd_ref[...]
        h = jnp.maximum(
            jnp.dot(agg, w_ref[...], preferred_element_type=jnp.float32)
            + b_ref[...],
            0.0,
        )
        hs = h * ns_ref[...]
        hs_ref[0] = hs[:, : H // 2]
        hs_ref[1] = hs[:, H // 2 :]

    grid = N // BN
    return pl.pallas_call(
        body,
        grid=(grid,),
        in_specs=[
            pl.BlockSpec((NC, BN, D), lambda i: (0, i, 0)),
            pl.BlockSpec((BN, 1), lambda i: (i, 0)),
            pl.BlockSpec((BN, 1), lambda i: (i, 0)),
            pl.BlockSpec((D, H), lambda i: (0, 0)),
            pl.BlockSpec((H,), lambda i: (0,)),
        ],
        out_specs=pl.BlockSpec((NC, BN, H // 2), lambda i: (0, i, 0)),
        out_shape=jax.ShapeDtypeStruct((NC, N, H // 2), jnp.float32),
    )(aggp, nd, ns, g1W, g1b)


def _ln(x, g, b):
    m = jnp.mean(x, axis=-1, keepdims=True)
    v = jnp.mean(jnp.square(x - m), axis=-1, keepdims=True)
    return (x - m) * lax.rsqrt(v + EPS) * g + b


def _tc_mlp(agg2p, nd, ws):
    (g2W, g2b, e1W, e1b, e1g, e1be, e2W, e2b, e2g, e2be,
     d1W, d1b, d1g, d1be, d2W, d2b, d2g, d2be,
     m1W, m1b, m1g, m1be, m2W, m2b) = ws

    def body(agg_ref, nd_ref, g2W_, g2b_, e1W_, e1b_, e1g_, e1be_, e2W_,
             e2b_, e2g_, e2be_, d1W_, d1b_, d1g_, d1be_, d2W_, d2b_, d2g_,
             d2be_, m1W_, m1b_, m1g_, m1be_, m2W_, m2b_, out_ref):
        agg = jnp.concatenate([agg_ref[0], agg_ref[1]], axis=1)
        agg = agg * nd_ref[...]

        def mm(a, w, b):
            return jnp.dot(a, w[...], preferred_element_type=jnp.float32) + b[...]

        h = jnp.maximum(mm(agg, g2W_, g2b_), 0.0)
        h = _ln(jnp.maximum(mm(h, e1W_, e1b_), 0.0), e1g_[...], e1be_[...])
        z = _ln(jnp.maximum(mm(h, e2W_, e2b_), 0.0), e2g_[...], e2be_[...])
        h = _ln(jnp.maximum(mm(z, d1W_, d1b_), 0.0), d1g_[...], d1be_[...])
        h = _ln(jnp.maximum(mm(h, d2W_, d2b_), 0.0), d2g_[...], d2be_[...])
        h = _ln(jnp.maximum(mm(h, m1W_, m1b_), 0.0), m1g_[...], m1be_[...])
        out_ref[...] = jnp.maximum(mm(h, m2W_, m2b_), 0.0)

    grid = N // BN
    w_specs = []
    for warr in ws:
        if warr.ndim == 2:
            w_specs.append(pl.BlockSpec(warr.shape, lambda i: (0, 0)))
        else:
            w_specs.append(pl.BlockSpec(warr.shape, lambda i: (0,)))
    return pl.pallas_call(
        body,
        grid=(grid,),
        in_specs=[
            pl.BlockSpec((NC, BN, H // 2), lambda i: (0, i, 0)),
            pl.BlockSpec((BN, 1), lambda i: (i, 0)),
        ] + w_specs,
        out_specs=pl.BlockSpec((BN, OUT), lambda i: (i, 0)),
        out_shape=jax.ShapeDtypeStruct((N, OUT), jnp.float32),
    )(agg2p, nd, *ws)


def kernel(x, edge_index, g1W, g1b, g2W, g2b, e1W, e1b, e1g, e1be, e2W, e2b,
           e2g, e2be, d1W, d1b, d1g, d1be, d2W, d2b, d2g, d2be, m1W, m1b,
           m1g, m1be, m2W, m2b):
    src = edge_index[0]
    dst = edge_index[1]
    degp = _sc_degrees(src, dst).reshape(4, N, 1)
    ns, nd, xs = _tc_norm_scale(degp, x)
    zeros128 = jnp.zeros((N, 128), jnp.float32)
    edges1 = _pad_edges(src, dst, NW, 50)
    edges2 = _pad_edges(src, dst, NS, 100)
    aggp = _sc_message(xs, edges1, zeros128, False, 50)
    hs = _tc_gcn1(aggp, nd, ns, g1W, g1b)
    agg2p = _sc_message(hs, edges2, zeros128, True, 100)
    ws = (g2W, g2b, e1W, e1b, e1g, e1be, e2W, e2b, e2g, e2be,
          d1W, d1b, d1g, d1be, d2W, d2b, d2g, d2be,
          m1W, m1b, m1g, m1be, m2W, m2b)
    return _tc_mlp(agg2p, nd, ws)
